# baseline (speedup 1.0000x reference)
import jax
import jax.numpy as jnp
from jax import lax
from jax.experimental import pallas as pl
from jax.experimental.pallas import tpu as pltpu
from jax.experimental.pallas import tpu_sc as plsc

EMB = 32
BATCH = 16384
NC = 2
NS = 16
L = 16
NW = NC * NS
BPW = BATCH // NW
CH = 128
NCH = BPW // CH


def _body(uid_hbm, aid_hbm, uw_hbm, aw_hbm, out_hbm,
          uidx_v, aidx_v, urows_v, arows_v, scores_v, sem):
    wid = lax.axis_index("s") * NC + lax.axis_index("c")
    base = wid * BPW

    pltpu.sync_copy(uid_hbm.at[pl.ds(base, BPW)], uidx_v)
    pltpu.sync_copy(aid_hbm.at[pl.ds(base, BPW)], aidx_v)

    copies = []
    for c in range(NCH):
        copies.append(pltpu.async_copy(
            uw_hbm.at[uidx_v.at[pl.ds(c * CH, CH)]],
            urows_v.at[pl.ds(c * CH, CH)], sem))
        copies.append(pltpu.async_copy(
            aw_hbm.at[aidx_v.at[pl.ds(c * CH, CH)]],
            arows_v.at[pl.ds(c * CH, CH)], sem))
    for cp in copies:
        cp.wait()

    lanes = lax.iota(jnp.int32, L)

    def group(g, carry):
        row = lanes + g * L
        acc = jnp.zeros((L,), jnp.float32)
        for j in range(EMB):
            col = jnp.full((L,), j, jnp.int32)
            u = plsc.load_gather(urows_v, [row, col])
            a = plsc.load_gather(arows_v, [row, col])
            acc = acc + u * a
        scores_v[pl.ds(g * L, L)] = acc
        return carry

    lax.fori_loop(0, BPW // L, group, 0)

    pltpu.sync_copy(scores_v, out_hbm.at[pl.ds(base, BPW)])


@jax.jit
def kernel(user_ids, anime_ids, user_emb_w, anime_emb_w):
    mesh = plsc.VectorSubcoreMesh(core_axis_name="c", subcore_axis_name="s")
    run = pl.kernel(
        _body,
        out_type=jax.ShapeDtypeStruct((BATCH,), jnp.float32),
        mesh=mesh,
        compiler_params=pltpu.CompilerParams(
            needs_layout_passes=False, use_tc_tiling_on_sc=False),
        scratch_types=[
            pltpu.VMEM((BPW,), jnp.int32),
            pltpu.VMEM((BPW,), jnp.int32),
            pltpu.VMEM((BPW, EMB), jnp.float32),
            pltpu.VMEM((BPW, EMB), jnp.float32),
            pltpu.VMEM((BPW,), jnp.float32),
            pltpu.SemaphoreType.DMA,
        ],
    )
    return run(user_ids, anime_ids, user_emb_w, anime_emb_w)
